# Initial kernel scaffold; baseline (speedup 1.0000x reference)
import jax
import jax.numpy as jnp
from jax.experimental import pallas as pl


def kernel(anchors, deltas, scores):
    raise NotImplementedError("write your pallas kernel here")


# speedup vs baseline: 35.6278x; 35.6278x over previous
import functools

import jax
import jax.numpy as jnp
from jax import lax
from jax.experimental import pallas as pl
from jax.experimental.pallas import tpu as pltpu
from jax.experimental.pallas import tpu_sc as plsc
import numpy as np

_N = 20000
_NP = 20480
_ROWS = 160
_CROWS = 96
_PRE = 12000
_POST = 2000
_ORECS = 16
_THRESH = 0.7
_IMG_W = 1024.0
_IMG_H = 1024.0
_SCALE_CLAMP = float(np.log(1000.0 / 16.0))


def _stage_a_body(a_ref, d_ref, s_ref, rank_ref, x1_ref, y1_ref, x2_ref,
                  y2_ref, ms_ref):
    ax1, ay1, ax2, ay2 = a_ref[0], a_ref[1], a_ref[2], a_ref[3]
    widths = ax2 - ax1
    heights = ay2 - ay1
    ctr_x = ax1 + 0.5 * widths
    ctr_y = ay1 + 0.5 * heights
    dx, dy = d_ref[0], d_ref[1]
    dw = jnp.minimum(d_ref[2], _SCALE_CLAMP)
    dh = jnp.minimum(d_ref[3], _SCALE_CLAMP)
    pred_ctr_x = dx * widths + ctr_x
    pred_ctr_y = dy * heights + ctr_y
    pred_w = jnp.exp(dw) * widths
    pred_h = jnp.exp(dh) * heights
    x1 = jnp.clip(pred_ctr_x - 0.5 * pred_w, 0.0, _IMG_W)
    y1 = jnp.clip(pred_ctr_y - 0.5 * pred_h, 0.0, _IMG_H)
    x2 = jnp.clip(pred_ctr_x + 0.5 * pred_w, 0.0, _IMG_W)
    y2 = jnp.clip(pred_ctr_y + 0.5 * pred_h, 0.0, _IMG_H)
    valid = ((x2 - x1) > 0.0) & ((y2 - y1) > 0.0)
    ms = jnp.where(valid, s_ref[...], -jnp.inf)
    x1_ref[...] = x1
    y1_ref[...] = y1
    x2_ref[...] = x2
    y2_ref[...] = y2
    ms_ref[...] = ms

    def outer(r, carry):
        trow = ms_ref[pl.ds(r, 1), :]
        tcol = jnp.transpose(trow)
        tb = jnp.broadcast_to(tcol, (128, 128))

        def body_ge(i, acc):
            cb = jnp.broadcast_to(ms_ref[pl.ds(i, 1), :], (128, 128))
            return acc + (cb >= tb).astype(jnp.float32)

        def body_gt(i, acc):
            cb = jnp.broadcast_to(ms_ref[pl.ds(i, 1), :], (128, 128))
            return acc + (cb > tb).astype(jnp.float32)

        acc = jnp.zeros((128, 128), jnp.float32)
        acc = lax.fori_loop(0, r, body_ge, acc)
        acc = lax.fori_loop(r + 1, _ROWS, body_gt, acc)
        cb = jnp.broadcast_to(trow, (128, 128))
        lane_j = lax.broadcasted_iota(jnp.int32, (128, 128), 1)
        lane_t = lax.broadcasted_iota(jnp.int32, (128, 128), 0)
        acc = acc + (cb > tb).astype(jnp.float32)
        acc = acc + ((cb == tb) & (lane_j < lane_t)).astype(jnp.float32)
        cnt = jnp.sum(acc, axis=1, keepdims=True)
        rank_ref[pl.ds(r, 1), :] = jnp.transpose(cnt).astype(jnp.int32)
        return carry

    lax.fori_loop(0, _ROWS, outer, 0)


def _stage_a(anchors_t, deltas_t, scores_p):
    return pl.pallas_call(
        _stage_a_body,
        out_shape=(
            jax.ShapeDtypeStruct((_ROWS, 128), jnp.int32),
            jax.ShapeDtypeStruct((_ROWS, 128), jnp.float32),
            jax.ShapeDtypeStruct((_ROWS, 128), jnp.float32),
            jax.ShapeDtypeStruct((_ROWS, 128), jnp.float32),
            jax.ShapeDtypeStruct((_ROWS, 128), jnp.float32),
            jax.ShapeDtypeStruct((_ROWS, 128), jnp.float32),
        ),
    )(anchors_t, deltas_t, scores_p)


def _make_sc_scatter():
    mesh = plsc.VectorSubcoreMesh(core_axis_name="c", subcore_axis_name="s")
    n_cores = 2
    rows_per_worker = 8
    n_workers = _ROWS // rows_per_worker

    @functools.partial(
        pl.kernel,
        mesh=mesh,
        out_type=[jax.ShapeDtypeStruct((_NP,), jnp.float32)
                  for _ in range(5)],
        scratch_types=(
            [pltpu.VMEM((rows_per_worker, 128), jnp.int32)]
            + [pltpu.VMEM((rows_per_worker, 128), jnp.float32)
               for _ in range(5)]
            + [pltpu.SemaphoreType.DMA]
        ),
    )
    def sc_scatter(rank_hbm, x1h, y1h, x2h, y2h, msh,
                   o0, o1, o2, o3, o4,
                   idx_v, v0, v1, v2, v3, v4, sem):
        w = lax.axis_index("s") * n_cores + lax.axis_index("c")

        @pl.when(w < n_workers)
        def _():
            base = w * rows_per_worker
            pltpu.sync_copy(rank_hbm.at[pl.ds(base, rows_per_worker), :],
                            idx_v)
            ins = [x1h, y1h, x2h, y2h, msh]
            stages = [v0, v1, v2, v3, v4]
            outs = [o0, o1, o2, o3, o4]
            for c in range(5):
                pltpu.sync_copy(ins[c].at[pl.ds(base, rows_per_worker), :],
                                stages[c])
            descs = []
            for c in range(5):
                for j in range(rows_per_worker):
                    descs.append(
                        pltpu.async_copy(stages[c].at[j],
                                         outs[c].at[idx_v.at[j]], sem))
            for d in descs:
                d.wait()

    return sc_scatter


def _iou_tile(cx1, cy1, cx2, cy2, carea, rx1, ry1, rx2, ry2, rarea):
    ix1 = jnp.maximum(rx1, cx1)
    iy1 = jnp.maximum(ry1, cy1)
    ix2 = jnp.minimum(rx2, cx2)
    iy2 = jnp.minimum(ry2, cy2)
    iw = jnp.maximum(ix2 - ix1, 0.0)
    ih = jnp.maximum(iy2 - iy1, 0.0)
    inter = iw * ih
    return inter / (rarea + carea - inter + 1e-9)


def _stage_c_body(sx1_ref, sy1_ref, sx2_ref, sy2_ref, ss_ref,
                  o0_ref, o1_ref, o2_ref, o3_ref, o4_ref,
                  keep_ref, sc_ref, area_ref):
    big = (_CROWS, 128)
    row_i = lax.broadcasted_iota(jnp.int32, big, 0)
    lane_i = lax.broadcasted_iota(jnp.int32, big, 1)
    flat = row_i * 128 + lane_i
    sc_ref[...] = jnp.where(flat < _PRE, ss_ref[...], -jnp.inf)
    x1a, y1a = sx1_ref[...], sy1_ref[...]
    x2a, y2a = sx2_ref[...], sy2_ref[...]
    area_ref[...] = (jnp.maximum(x2a - x1a, 0.0)
                     * jnp.maximum(y2a - y1a, 0.0))
    keep_ref[...] = jnp.zeros(big, jnp.float32)

    def load_row(ref, i):
        return ref[pl.ds(i, 1), :]

    def block_body(carry):
        b, cnt = carry
        bx1 = load_row(sx1_ref, b)
        by1 = load_row(sy1_ref, b)
        bx2 = load_row(sx2_ref, b)
        by2 = load_row(sy2_ref, b)
        barea = load_row(area_ref, b)
        bs = load_row(sc_ref, b)
        cx1 = jnp.broadcast_to(jnp.transpose(bx1), (128, 128))
        cy1 = jnp.broadcast_to(jnp.transpose(by1), (128, 128))
        cx2 = jnp.broadcast_to(jnp.transpose(bx2), (128, 128))
        cy2 = jnp.broadcast_to(jnp.transpose(by2), (128, 128))
        carea = jnp.broadcast_to(jnp.transpose(barea), (128, 128))
        scol = jnp.transpose(bs)

        def prior(p, acc):
            rx1 = jnp.broadcast_to(load_row(sx1_ref, p), (128, 128))
            ry1 = jnp.broadcast_to(load_row(sy1_ref, p), (128, 128))
            rx2 = jnp.broadcast_to(load_row(sx2_ref, p), (128, 128))
            ry2 = jnp.broadcast_to(load_row(sy2_ref, p), (128, 128))
            rarea = jnp.broadcast_to(load_row(area_ref, p), (128, 128))
            kp = jnp.broadcast_to(load_row(keep_ref, p), (128, 128))
            iou = _iou_tile(cx1, cy1, cx2, cy2, carea,
                            rx1, ry1, rx2, ry2, rarea)
            return jnp.maximum(
                acc, jnp.where((iou >= _THRESH) & (kp > 0.0), 1.0, 0.0))

        sup0 = lax.fori_loop(0, b, prior,
                             jnp.zeros((128, 128), jnp.float32))
        supped = jnp.max(sup0, axis=1, keepdims=True) > 0.0
        valid = jnp.where((scol > -jnp.inf) & (~supped), 1.0, 0.0)

        rx1 = jnp.broadcast_to(bx1, (128, 128))
        ry1 = jnp.broadcast_to(by1, (128, 128))
        rx2 = jnp.broadcast_to(bx2, (128, 128))
        ry2 = jnp.broadcast_to(by2, (128, 128))
        rarea = jnp.broadcast_to(barea, (128, 128))
        iou_bb = _iou_tile(cx1, cy1, cx2, cy2, carea,
                           rx1, ry1, rx2, ry2, rarea)
        lane_u = lax.broadcasted_iota(jnp.int32, (128, 128), 1)
        lane_e = lax.broadcasted_iota(jnp.int32, (128, 128), 0)
        s_mat = jnp.where((iou_bb >= _THRESH) & (lane_u < lane_e), 1.0, 0.0)

        def fix_cond(c):
            _, changed, it = c
            return changed & (it < 130)

        def fix_body(c):
            k, _, it = c
            kb = jnp.broadcast_to(jnp.transpose(k), (128, 128))
            sup = jnp.max(s_mat * kb, axis=1, keepdims=True)
            k_new = jnp.where(sup > 0.0, 0.0, valid)
            changed = jnp.sum(jnp.abs(k_new - k)) > 0.0
            return k_new, changed, it + 1

        k_fin, _, _ = lax.while_loop(
            fix_cond, fix_body,
            (valid, jnp.bool_(True), jnp.int32(0)))

        keep_ref[pl.ds(b, 1), :] = jnp.transpose(k_fin)
        return b + 1, cnt + jnp.sum(k_fin)

    def block_cond(carry):
        b, cnt = carry
        return (b < _CROWS) & (cnt < float(_POST))

    _, total = lax.while_loop(block_cond, block_body,
                              (jnp.int32(0), jnp.float32(0.0)))

    keep = keep_ref[...]
    lane_pre = keep
    for sh in (1, 2, 4, 8, 16, 32, 64):
        rolled = jnp.roll(lane_pre, sh, axis=1)
        lane_pre = lane_pre + jnp.where(lane_i >= sh, rolled, 0.0)
    rowsum = jnp.broadcast_to(jnp.sum(keep, axis=1, keepdims=True), big)
    rowcum = rowsum
    for sh in (1, 2, 4, 8, 16, 32, 64):
        rolled = jnp.roll(rowcum, sh, axis=0)
        rowcum = rowcum + jnp.where(row_i >= sh, rolled, 0.0)
    pos = (rowcum - rowsum) + lane_pre - keep
    area_ref[...] = jnp.where(keep > 0.0, pos, -1.0)

    pad0 = [jnp.broadcast_to(r[0:1, 0:1], (128, 1))
            for r in (sx1_ref, sy1_ref, sx2_ref, sy2_ref)]
    out_refs = [o0_ref, o1_ref, o2_ref, o3_ref, o4_ref]
    pay_refs = [sx1_ref, sy1_ref, sx2_ref, sy2_ref, sc_ref]
    sub_i = lax.broadcasted_iota(jnp.int32, (128, 128), 0)
    totb = jnp.full((128, 1), 0.0) + total
    for oc in range(_ORECS):
        r_col = (oc * 128 + sub_i).astype(jnp.float32)

        def omatch(j, accs):
            pe = jnp.broadcast_to(area_ref[pl.ds(j, 1), :], (128, 128))
            m = pe == r_col
            new = []
            for a, pref in zip(accs, pay_refs):
                pb = jnp.broadcast_to(pref[pl.ds(j, 1), :], (128, 128))
                new.append(a + jnp.where(m, pb, 0.0))
            return tuple(new)

        accs = tuple(jnp.zeros((128, 128), jnp.float32) for _ in range(5))
        accs = lax.fori_loop(0, _CROWS, omatch, accs)
        r0 = (oc * 128 + lax.broadcasted_iota(
            jnp.int32, (128, 1), 0)).astype(jnp.float32)
        have = r0 < totb
        for c in range(5):
            col = jnp.sum(accs[c], axis=1, keepdims=True)
            pad = pad0[c] if c < 4 else jnp.full((128, 1), -jnp.inf)
            val = jnp.where(have, col, pad)
            out_refs[c][pl.ds(oc, 1), :] = jnp.transpose(val)


def _stage_c(sx1, sy1, sx2, sy2, ss):
    blk = pl.BlockSpec((_CROWS, 128), lambda i: (0, 0))
    oblk = pl.BlockSpec((_ORECS, 128), lambda i: (0, 0))
    return pl.pallas_call(
        _stage_c_body,
        grid=(1,),
        in_specs=[blk] * 5,
        out_specs=[oblk] * 5,
        out_shape=[jax.ShapeDtypeStruct((_ORECS, 128), jnp.float32)
                   for _ in range(5)],
        scratch_shapes=[pltpu.VMEM((_CROWS, 128), jnp.float32)
                        for _ in range(3)],
    )(sx1, sy1, sx2, sy2, ss)


def kernel(anchors, deltas, scores):
    pad = _NP - _N
    a_p = jnp.pad(anchors, ((0, pad), (0, 0)))
    d_p = jnp.pad(deltas, ((0, pad), (0, 0)))
    s_p = jnp.pad(scores, (0, pad))
    a_t = jnp.transpose(a_p).reshape(4, _ROWS, 128)
    d_t = jnp.transpose(d_p).reshape(4, _ROWS, 128)
    s_2d = s_p.reshape(_ROWS, 128)

    rank, x1, y1, x2, y2, ms = _stage_a(a_t, d_t, s_2d)

    sc_scatter = _make_sc_scatter()
    s0, s1, s2, s3, s4 = sc_scatter(rank, x1, y1, x2, y2, ms)

    outs = _stage_c(s0.reshape(_ROWS, 128), s1.reshape(_ROWS, 128),
                    s2.reshape(_ROWS, 128), s3.reshape(_ROWS, 128),
                    s4.reshape(_ROWS, 128))
    cols = [o.reshape(-1)[:_POST] for o in outs]
    return jnp.stack(cols, axis=1)


# speedup vs baseline: 56.0395x; 1.5729x over previous
import functools

import jax
import jax.numpy as jnp
from jax import lax
from jax.experimental import pallas as pl
from jax.experimental.pallas import tpu as pltpu
from jax.experimental.pallas import tpu_sc as plsc
import numpy as np

_N = 20000
_NP = 20480
_ROWS = 160
_CROWS = 96
_PRE = 12000
_POST = 2000
_ORECS = 16
_THRESH = 0.7
_IMG_W = 1024.0
_IMG_H = 1024.0
_SCALE_CLAMP = float(np.log(1000.0 / 16.0))


def _stage_a_body(a_ref, d_ref, s_ref, rank_ref, x1_ref, y1_ref, x2_ref,
                  y2_ref, ms_ref):
    ax1, ay1, ax2, ay2 = a_ref[0], a_ref[1], a_ref[2], a_ref[3]
    widths = ax2 - ax1
    heights = ay2 - ay1
    ctr_x = ax1 + 0.5 * widths
    ctr_y = ay1 + 0.5 * heights
    dx, dy = d_ref[0], d_ref[1]
    dw = jnp.minimum(d_ref[2], _SCALE_CLAMP)
    dh = jnp.minimum(d_ref[3], _SCALE_CLAMP)
    pred_ctr_x = dx * widths + ctr_x
    pred_ctr_y = dy * heights + ctr_y
    pred_w = jnp.exp(dw) * widths
    pred_h = jnp.exp(dh) * heights
    x1 = jnp.clip(pred_ctr_x - 0.5 * pred_w, 0.0, _IMG_W)
    y1 = jnp.clip(pred_ctr_y - 0.5 * pred_h, 0.0, _IMG_H)
    x2 = jnp.clip(pred_ctr_x + 0.5 * pred_w, 0.0, _IMG_W)
    y2 = jnp.clip(pred_ctr_y + 0.5 * pred_h, 0.0, _IMG_H)
    valid = ((x2 - x1) > 0.0) & ((y2 - y1) > 0.0)
    ms = jnp.where(valid, s_ref[...], -jnp.inf)
    x1_ref[...] = x1
    y1_ref[...] = y1
    x2_ref[...] = x2
    y2_ref[...] = y2
    ms_ref[...] = ms

    def outer(r, carry):
        trow = ms_ref[pl.ds(r, 1), :]
        tcol = jnp.transpose(trow)
        tb = jnp.broadcast_to(tcol, (128, 128))

        def body_ge(i, acc):
            cb = jnp.broadcast_to(ms_ref[pl.ds(i, 1), :], (128, 128))
            return acc + (cb >= tb).astype(jnp.float32)

        def body_gt(i, acc):
            cb = jnp.broadcast_to(ms_ref[pl.ds(i, 1), :], (128, 128))
            return acc + (cb > tb).astype(jnp.float32)

        acc = jnp.zeros((128, 128), jnp.float32)
        acc = lax.fori_loop(0, r, body_ge, acc)
        acc = lax.fori_loop(r + 1, _ROWS, body_gt, acc)
        cb = jnp.broadcast_to(trow, (128, 128))
        lane_j = lax.broadcasted_iota(jnp.int32, (128, 128), 1)
        lane_t = lax.broadcasted_iota(jnp.int32, (128, 128), 0)
        acc = acc + (cb > tb).astype(jnp.float32)
        acc = acc + ((cb == tb) & (lane_j < lane_t)).astype(jnp.float32)
        cnt = jnp.sum(acc, axis=1, keepdims=True)
        rank_ref[pl.ds(r, 1), :] = jnp.transpose(cnt).astype(jnp.int32)
        return carry

    lax.fori_loop(0, _ROWS, outer, 0)


def _stage_a(anchors_t, deltas_t, scores_p):
    return pl.pallas_call(
        _stage_a_body,
        out_shape=(
            jax.ShapeDtypeStruct((_ROWS, 128), jnp.int32),
            jax.ShapeDtypeStruct((_ROWS, 128), jnp.float32),
            jax.ShapeDtypeStruct((_ROWS, 128), jnp.float32),
            jax.ShapeDtypeStruct((_ROWS, 128), jnp.float32),
            jax.ShapeDtypeStruct((_ROWS, 128), jnp.float32),
            jax.ShapeDtypeStruct((_ROWS, 128), jnp.float32),
        ),
    )(anchors_t, deltas_t, scores_p)


def _make_sc_scatter():
    mesh = plsc.VectorSubcoreMesh(core_axis_name="c", subcore_axis_name="s")
    chunk_rows = 8
    n_chunks = _ROWS // chunk_rows
    out_elems = _NP // 16

    @functools.partial(
        pl.kernel,
        mesh=mesh,
        out_type=[jax.ShapeDtypeStruct((_NP,), jnp.float32)
                  for _ in range(5)],
        scratch_types=(
            [pltpu.VMEM((chunk_rows, 128), jnp.int32)]
            + [pltpu.VMEM((chunk_rows, 128), jnp.float32)
               for _ in range(5)]
            + [pltpu.VMEM_SHARED((_NP,), jnp.float32) for _ in range(5)]
            + [pltpu.SemaphoreType.DMA]
        ),
    )
    def sc_scatter(rank_hbm, x1h, y1h, x2h, y2h, msh,
                   o0, o1, o2, o3, o4,
                   idx_v, v0, v1, v2, v3, v4,
                   sh0, sh1, sh2, sh3, sh4, sem):
        core = lax.axis_index("c")
        tec = lax.axis_index("s")
        ins = [x1h, y1h, x2h, y2h, msh]
        stages = [v0, v1, v2, v3, v4]
        shared = [sh0, sh1, sh2, sh3, sh4]
        outs = [o0, o1, o2, o3, o4]

        @pl.when(core == 0)
        def _():
            def do_chunk(k):
                base = k * chunk_rows
                pltpu.sync_copy(rank_hbm.at[pl.ds(base, chunk_rows), :],
                                idx_v)
                for c in range(5):
                    pltpu.sync_copy(ins[c].at[pl.ds(base, chunk_rows), :],
                                    stages[c])
                descs = []
                for c in range(5):
                    for j in range(chunk_rows):
                        descs.append(
                            pltpu.async_copy(stages[c].at[j],
                                             shared[c].at[idx_v.at[j]],
                                             sem))
                for d in descs:
                    d.wait()

            do_chunk(tec)

            @pl.when(tec < n_chunks - 16)
            def _():
                do_chunk(tec + 16)

        plsc.subcore_barrier()

        @pl.when(core == 0)
        def _():
            base = tec * out_elems
            for c in range(5):
                pltpu.sync_copy(shared[c].at[pl.ds(base, out_elems)],
                                outs[c].at[pl.ds(base, out_elems)])

    return sc_scatter


def _iou_tile(cx1, cy1, cx2, cy2, carea, rx1, ry1, rx2, ry2, rarea):
    ix1 = jnp.maximum(rx1, cx1)
    iy1 = jnp.maximum(ry1, cy1)
    ix2 = jnp.minimum(rx2, cx2)
    iy2 = jnp.minimum(ry2, cy2)
    iw = jnp.maximum(ix2 - ix1, 0.0)
    ih = jnp.maximum(iy2 - iy1, 0.0)
    inter = iw * ih
    return inter / (rarea + carea - inter + 1e-9)


def _stage_c_body(sx1_ref, sy1_ref, sx2_ref, sy2_ref, ss_ref,
                  o0_ref, o1_ref, o2_ref, o3_ref, o4_ref,
                  keep_ref, sc_ref, area_ref):
    big = (_CROWS, 128)
    row_i = lax.broadcasted_iota(jnp.int32, big, 0)
    lane_i = lax.broadcasted_iota(jnp.int32, big, 1)
    flat = row_i * 128 + lane_i
    sc_ref[...] = jnp.where(flat < _PRE, ss_ref[...], -jnp.inf)
    x1a, y1a = sx1_ref[...], sy1_ref[...]
    x2a, y2a = sx2_ref[...], sy2_ref[...]
    area_ref[...] = (jnp.maximum(x2a - x1a, 0.0)
                     * jnp.maximum(y2a - y1a, 0.0))
    keep_ref[...] = jnp.zeros(big, jnp.float32)

    def load_row(ref, i):
        return ref[pl.ds(i, 1), :]

    def block_body(carry):
        b, cnt = carry
        bx1 = load_row(sx1_ref, b)
        by1 = load_row(sy1_ref, b)
        bx2 = load_row(sx2_ref, b)
        by2 = load_row(sy2_ref, b)
        barea = load_row(area_ref, b)
        bs = load_row(sc_ref, b)
        cx1 = jnp.broadcast_to(jnp.transpose(bx1), (128, 128))
        cy1 = jnp.broadcast_to(jnp.transpose(by1), (128, 128))
        cx2 = jnp.broadcast_to(jnp.transpose(bx2), (128, 128))
        cy2 = jnp.broadcast_to(jnp.transpose(by2), (128, 128))
        carea = jnp.broadcast_to(jnp.transpose(barea), (128, 128))
        scol = jnp.transpose(bs)

        def prior(p, acc):
            rx1 = jnp.broadcast_to(load_row(sx1_ref, p), (128, 128))
            ry1 = jnp.broadcast_to(load_row(sy1_ref, p), (128, 128))
            rx2 = jnp.broadcast_to(load_row(sx2_ref, p), (128, 128))
            ry2 = jnp.broadcast_to(load_row(sy2_ref, p), (128, 128))
            rarea = jnp.broadcast_to(load_row(area_ref, p), (128, 128))
            kp = jnp.broadcast_to(load_row(keep_ref, p), (128, 128))
            iou = _iou_tile(cx1, cy1, cx2, cy2, carea,
                            rx1, ry1, rx2, ry2, rarea)
            return jnp.maximum(
                acc, jnp.where((iou >= _THRESH) & (kp > 0.0), 1.0, 0.0))

        sup0 = lax.fori_loop(0, b, prior,
                             jnp.zeros((128, 128), jnp.float32))
        supped = jnp.max(sup0, axis=1, keepdims=True) > 0.0
        valid = jnp.where((scol > -jnp.inf) & (~supped), 1.0, 0.0)

        rx1 = jnp.broadcast_to(bx1, (128, 128))
        ry1 = jnp.broadcast_to(by1, (128, 128))
        rx2 = jnp.broadcast_to(bx2, (128, 128))
        ry2 = jnp.broadcast_to(by2, (128, 128))
        rarea = jnp.broadcast_to(barea, (128, 128))
        iou_bb = _iou_tile(cx1, cy1, cx2, cy2, carea,
                           rx1, ry1, rx2, ry2, rarea)
        lane_u = lax.broadcasted_iota(jnp.int32, (128, 128), 1)
        lane_e = lax.broadcasted_iota(jnp.int32, (128, 128), 0)
        s_mat = jnp.where((iou_bb >= _THRESH) & (lane_u < lane_e), 1.0, 0.0)

        def fix_cond(c):
            _, changed, it = c
            return changed & (it < 130)

        def fix_body(c):
            k, _, it = c
            kb = jnp.broadcast_to(jnp.transpose(k), (128, 128))
            sup = jnp.max(s_mat * kb, axis=1, keepdims=True)
            k_new = jnp.where(sup > 0.0, 0.0, valid)
            changed = jnp.sum(jnp.abs(k_new - k)) > 0.0
            return k_new, changed, it + 1

        k_fin, _, _ = lax.while_loop(
            fix_cond, fix_body,
            (valid, jnp.bool_(True), jnp.int32(0)))

        keep_ref[pl.ds(b, 1), :] = jnp.transpose(k_fin)
        return b + 1, cnt + jnp.sum(k_fin)

    def block_cond(carry):
        b, cnt = carry
        return (b < _CROWS) & (cnt < float(_POST))

    b_fin, total = lax.while_loop(block_cond, block_body,
                                  (jnp.int32(0), jnp.float32(0.0)))

    keep = keep_ref[...]
    lane_pre = keep
    for sh in (1, 2, 4, 8, 16, 32, 64):
        rolled = jnp.roll(lane_pre, sh, axis=1)
        lane_pre = lane_pre + jnp.where(lane_i >= sh, rolled, 0.0)
    rowsum = jnp.broadcast_to(jnp.sum(keep, axis=1, keepdims=True), big)
    rowcum = rowsum
    for sh in (1, 2, 4, 8, 16, 32, 64):
        rolled = jnp.roll(rowcum, sh, axis=0)
        rowcum = rowcum + jnp.where(row_i >= sh, rolled, 0.0)
    pos = (rowcum - rowsum) + lane_pre - keep
    area_ref[...] = jnp.where(keep > 0.0, pos, -1.0)

    pad0 = [jnp.broadcast_to(r[0:1, 0:1], (128, 1))
            for r in (sx1_ref, sy1_ref, sx2_ref, sy2_ref)]
    out_refs = [o0_ref, o1_ref, o2_ref, o3_ref, o4_ref]
    pay_refs = [sx1_ref, sy1_ref, sx2_ref, sy2_ref, sc_ref]
    sub_i = lax.broadcasted_iota(jnp.int32, (128, 128), 0)
    totb = jnp.full((128, 1), 0.0) + total
    for oc in range(_ORECS):
        r_col = (oc * 128 + sub_i).astype(jnp.float32)

        def omatch(j, accs):
            pe = jnp.broadcast_to(area_ref[pl.ds(j, 1), :], (128, 128))
            m = pe == r_col
            new = []
            for a, pref in zip(accs, pay_refs):
                pb = jnp.broadcast_to(pref[pl.ds(j, 1), :], (128, 128))
                new.append(a + jnp.where(m, pb, 0.0))
            return tuple(new)

        accs = tuple(jnp.zeros((128, 128), jnp.float32) for _ in range(5))
        accs = lax.fori_loop(0, b_fin, omatch, accs)
        r0 = (oc * 128 + lax.broadcasted_iota(
            jnp.int32, (128, 1), 0)).astype(jnp.float32)
        have = r0 < totb
        for c in range(5):
            col = jnp.sum(accs[c], axis=1, keepdims=True)
            pad = pad0[c] if c < 4 else jnp.full((128, 1), -jnp.inf)
            val = jnp.where(have, col, pad)
            out_refs[c][pl.ds(oc, 1), :] = jnp.transpose(val)


def _stage_c(sx1, sy1, sx2, sy2, ss):
    blk = pl.BlockSpec((_CROWS, 128), lambda i: (0, 0))
    oblk = pl.BlockSpec((_ORECS, 128), lambda i: (0, 0))
    return pl.pallas_call(
        _stage_c_body,
        grid=(1,),
        in_specs=[blk] * 5,
        out_specs=[oblk] * 5,
        out_shape=[jax.ShapeDtypeStruct((_ORECS, 128), jnp.float32)
                   for _ in range(5)],
        scratch_shapes=[pltpu.VMEM((_CROWS, 128), jnp.float32)
                        for _ in range(3)],
    )(sx1, sy1, sx2, sy2, ss)


def kernel(anchors, deltas, scores):
    pad = _NP - _N
    a_p = jnp.pad(anchors, ((0, pad), (0, 0)))
    d_p = jnp.pad(deltas, ((0, pad), (0, 0)))
    s_p = jnp.pad(scores, (0, pad))
    a_t = jnp.transpose(a_p).reshape(4, _ROWS, 128)
    d_t = jnp.transpose(d_p).reshape(4, _ROWS, 128)
    s_2d = s_p.reshape(_ROWS, 128)

    rank, x1, y1, x2, y2, ms = _stage_a(a_t, d_t, s_2d)

    sc_scatter = _make_sc_scatter()
    s0, s1, s2, s3, s4 = sc_scatter(rank, x1, y1, x2, y2, ms)

    outs = _stage_c(s0.reshape(_ROWS, 128), s1.reshape(_ROWS, 128),
                    s2.reshape(_ROWS, 128), s3.reshape(_ROWS, 128),
                    s4.reshape(_ROWS, 128))
    cols = [o.reshape(-1)[:_POST] for o in outs]
    return jnp.stack(cols, axis=1)


# speedup vs baseline: 96.4044x; 1.7203x over previous
import functools

import jax
import jax.numpy as jnp
from jax import lax
from jax.experimental import pallas as pl
from jax.experimental.pallas import tpu as pltpu
from jax.experimental.pallas import tpu_sc as plsc
import numpy as np

_N = 20000
_NP = 20480
_ROWS = 160
_CROWS = 96
_PRE = 12000
_POST = 2000
_ORECS = 16
_THRESH = 0.7
_IMG_W = 1024.0
_IMG_H = 1024.0
_SCALE_CLAMP = float(np.log(1000.0 / 16.0))


def _prefix_excl(x, rows):
    lane_i = lax.broadcasted_iota(jnp.int32, (rows, 128), 1)
    row_i = lax.broadcasted_iota(jnp.int32, (rows, 128), 0)
    lpre = x
    for sh in (1, 2, 4, 8, 16, 32, 64):
        lpre = lpre + jnp.where(lane_i >= sh, jnp.roll(lpre, sh, axis=1), 0.0)
    rowsum = jnp.broadcast_to(jnp.sum(x, axis=1, keepdims=True), (rows, 128))
    rcum = rowsum
    sh = 1
    while sh < rows:
        rcum = rcum + jnp.where(row_i >= sh, jnp.roll(rcum, sh, axis=0), 0.0)
        sh *= 2
    return (rcum - rowsum) + (lpre - x)


def _stage_a_body(a_ref, d_ref, s_ref, dest_ref, x1_ref, y1_ref, x2_ref,
                  y2_ref, ms_ref):
    ax1, ay1, ax2, ay2 = a_ref[0], a_ref[1], a_ref[2], a_ref[3]
    widths = ax2 - ax1
    heights = ay2 - ay1
    ctr_x = ax1 + 0.5 * widths
    ctr_y = ay1 + 0.5 * heights
    dx, dy = d_ref[0], d_ref[1]
    dw = jnp.minimum(d_ref[2], _SCALE_CLAMP)
    dh = jnp.minimum(d_ref[3], _SCALE_CLAMP)
    pred_ctr_x = dx * widths + ctr_x
    pred_ctr_y = dy * heights + ctr_y
    pred_w = jnp.exp(dw) * widths
    pred_h = jnp.exp(dh) * heights
    x1 = jnp.clip(pred_ctr_x - 0.5 * pred_w, 0.0, _IMG_W)
    y1 = jnp.clip(pred_ctr_y - 0.5 * pred_h, 0.0, _IMG_H)
    x2 = jnp.clip(pred_ctr_x + 0.5 * pred_w, 0.0, _IMG_W)
    y2 = jnp.clip(pred_ctr_y + 0.5 * pred_h, 0.0, _IMG_H)
    valid = ((x2 - x1) > 0.0) & ((y2 - y1) > 0.0)
    ms = jnp.where(valid, s_ref[...], -jnp.inf)
    ms = jnp.where(ms == 0.0, 0.0, ms)
    x1_ref[...] = x1
    y1_ref[...] = y1
    x2_ref[...] = x2
    y2_ref[...] = y2
    ms_ref[...] = ms

    bits = lax.bitcast_convert_type(ms, jnp.uint32)
    neg = (bits >> jnp.uint32(31)) == jnp.uint32(1)
    uk = jnp.where(neg, ~bits, bits | jnp.uint32(0x80000000))

    def bis(i, kacc):
        kc = kacc | lax.shift_left(jnp.uint32(1),
                                   (31 - i).astype(jnp.uint32))
        cnt = jnp.sum(jnp.where(uk >= kc, 1.0, 0.0))
        return jnp.where(cnt >= float(_PRE), kc, kacc)

    kstar = lax.fori_loop(0, 32, bis, jnp.uint32(0))

    gt = uk > kstar
    eq = uk == kstar
    cnt_gt = jnp.sum(jnp.where(gt, 1.0, 0.0))
    need = float(_PRE) - cnt_gt
    eqpre = _prefix_excl(jnp.where(eq, 1.0, 0.0), _ROWS)
    cand = gt | (eq & (eqpre < need))
    candf = jnp.where(cand, 1.0, 0.0)
    pc = _prefix_excl(candf, _ROWS)
    row_i = lax.broadcasted_iota(jnp.int32, (_ROWS, 128), 0)
    lane_i = lax.broadcasted_iota(jnp.int32, (_ROWS, 128), 1)
    flat = (row_i * 128 + lane_i).astype(jnp.float32)
    dest = jnp.where(cand, pc, float(_PRE) + flat - pc)
    dest_ref[...] = dest.astype(jnp.int32)


def _stage_a(anchors_t, deltas_t, scores_p):
    return pl.pallas_call(
        _stage_a_body,
        out_shape=(
            jax.ShapeDtypeStruct((_ROWS, 128), jnp.int32),
            jax.ShapeDtypeStruct((_ROWS, 128), jnp.float32),
            jax.ShapeDtypeStruct((_ROWS, 128), jnp.float32),
            jax.ShapeDtypeStruct((_ROWS, 128), jnp.float32),
            jax.ShapeDtypeStruct((_ROWS, 128), jnp.float32),
            jax.ShapeDtypeStruct((_ROWS, 128), jnp.float32),
        ),
    )(anchors_t, deltas_t, scores_p)


def _stage_a2_body(ms_ref, rank_ref):
    def outer(r, carry):
        trow = ms_ref[pl.ds(r, 1), :]
        tb = jnp.broadcast_to(jnp.transpose(trow), (128, 128))

        def body_ge(i, acc):
            cb = jnp.broadcast_to(ms_ref[pl.ds(i, 1), :], (128, 128))
            return acc + (cb >= tb).astype(jnp.float32)

        def body_gt(i, acc):
            cb = jnp.broadcast_to(ms_ref[pl.ds(i, 1), :], (128, 128))
            return acc + (cb > tb).astype(jnp.float32)

        acc = jnp.zeros((128, 128), jnp.float32)
        acc = lax.fori_loop(0, r, body_ge, acc)
        acc = lax.fori_loop(r + 1, _CROWS, body_gt, acc)
        cb = jnp.broadcast_to(trow, (128, 128))
        lane_j = lax.broadcasted_iota(jnp.int32, (128, 128), 1)
        lane_t = lax.broadcasted_iota(jnp.int32, (128, 128), 0)
        acc = acc + (cb > tb).astype(jnp.float32)
        acc = acc + ((cb == tb) & (lane_j < lane_t)).astype(jnp.float32)
        cnt = jnp.sum(acc, axis=1, keepdims=True)
        rank_ref[pl.ds(r, 1), :] = jnp.transpose(cnt).astype(jnp.int32)
        return carry

    lax.fori_loop(0, _CROWS, outer, 0)


def _stage_a2(cms):
    blk = pl.BlockSpec((_CROWS, 128), lambda i: (0, 0))
    oblk = pl.BlockSpec((_CROWS, 128), lambda i: (0, 0))
    return pl.pallas_call(
        _stage_a2_body,
        grid=(1,),
        in_specs=[blk],
        out_specs=oblk,
        out_shape=jax.ShapeDtypeStruct((_CROWS, 128), jnp.int32),
    )(cms)


def _make_sc_scatter(n_rows):
    mesh = plsc.VectorSubcoreMesh(core_axis_name="c", subcore_axis_name="s")
    chunk_rows = 8
    n_chunks = n_rows // chunk_rows
    n_out = n_rows * 128
    out_elems = n_out // 16

    @functools.partial(
        pl.kernel,
        mesh=mesh,
        out_type=[jax.ShapeDtypeStruct((n_out,), jnp.float32)
                  for _ in range(5)],
        scratch_types=(
            [pltpu.VMEM((chunk_rows, 128), jnp.int32)]
            + [pltpu.VMEM((chunk_rows, 128), jnp.float32)
               for _ in range(5)]
            + [pltpu.VMEM_SHARED((n_out,), jnp.float32) for _ in range(5)]
            + [pltpu.SemaphoreType.DMA]
        ),
    )
    def sc_scatter(rank_hbm, x1h, y1h, x2h, y2h, msh,
                   o0, o1, o2, o3, o4,
                   idx_v, v0, v1, v2, v3, v4,
                   sh0, sh1, sh2, sh3, sh4, sem):
        core = lax.axis_index("c")
        tec = lax.axis_index("s")
        ins = [x1h, y1h, x2h, y2h, msh]
        stages = [v0, v1, v2, v3, v4]
        shared = [sh0, sh1, sh2, sh3, sh4]
        outs = [o0, o1, o2, o3, o4]

        @pl.when(core == 0)
        def _():
            def do_chunk(k):
                base = k * chunk_rows
                pltpu.sync_copy(rank_hbm.at[pl.ds(base, chunk_rows), :],
                                idx_v)
                for c in range(5):
                    pltpu.sync_copy(ins[c].at[pl.ds(base, chunk_rows), :],
                                    stages[c])
                descs = []
                for c in range(5):
                    for j in range(chunk_rows):
                        descs.append(
                            pltpu.async_copy(stages[c].at[j],
                                             shared[c].at[idx_v.at[j]],
                                             sem))
                for d in descs:
                    d.wait()

            @pl.when(tec < n_chunks)
            def _():
                do_chunk(tec)

            @pl.when(tec < n_chunks - 16)
            def _():
                do_chunk(tec + 16)

        plsc.subcore_barrier()

        @pl.when(core == 0)
        def _():
            base = tec * out_elems
            for c in range(5):
                pltpu.sync_copy(shared[c].at[pl.ds(base, out_elems)],
                                outs[c].at[pl.ds(base, out_elems)])

    return sc_scatter


def _iou_tile(cx1, cy1, cx2, cy2, carea, rx1, ry1, rx2, ry2, rarea):
    ix1 = jnp.maximum(rx1, cx1)
    iy1 = jnp.maximum(ry1, cy1)
    ix2 = jnp.minimum(rx2, cx2)
    iy2 = jnp.minimum(ry2, cy2)
    iw = jnp.maximum(ix2 - ix1, 0.0)
    ih = jnp.maximum(iy2 - iy1, 0.0)
    inter = iw * ih
    return inter / (rarea + carea - inter + 1e-9)


def _stage_c_body(sx1_ref, sy1_ref, sx2_ref, sy2_ref, ss_ref,
                  o0_ref, o1_ref, o2_ref, o3_ref, o4_ref,
                  keep_ref, sc_ref, area_ref):
    big = (_CROWS, 128)
    row_i = lax.broadcasted_iota(jnp.int32, big, 0)
    lane_i = lax.broadcasted_iota(jnp.int32, big, 1)
    flat = row_i * 128 + lane_i
    sc_ref[...] = jnp.where(flat < _PRE, ss_ref[...], -jnp.inf)
    x1a, y1a = sx1_ref[...], sy1_ref[...]
    x2a, y2a = sx2_ref[...], sy2_ref[...]
    area_ref[...] = (jnp.maximum(x2a - x1a, 0.0)
                     * jnp.maximum(y2a - y1a, 0.0))
    keep_ref[...] = jnp.zeros(big, jnp.float32)

    def load_row(ref, i):
        return ref[pl.ds(i, 1), :]

    def block_body(carry):
        b, cnt = carry
        bx1 = load_row(sx1_ref, b)
        by1 = load_row(sy1_ref, b)
        bx2 = load_row(sx2_ref, b)
        by2 = load_row(sy2_ref, b)
        barea = load_row(area_ref, b)
        bs = load_row(sc_ref, b)
        cx1 = jnp.broadcast_to(jnp.transpose(bx1), (128, 128))
        cy1 = jnp.broadcast_to(jnp.transpose(by1), (128, 128))
        cx2 = jnp.broadcast_to(jnp.transpose(bx2), (128, 128))
        cy2 = jnp.broadcast_to(jnp.transpose(by2), (128, 128))
        carea = jnp.broadcast_to(jnp.transpose(barea), (128, 128))
        scol = jnp.transpose(bs)

        def prior(p, acc):
            rx1 = jnp.broadcast_to(load_row(sx1_ref, p), (128, 128))
            ry1 = jnp.broadcast_to(load_row(sy1_ref, p), (128, 128))
            rx2 = jnp.broadcast_to(load_row(sx2_ref, p), (128, 128))
            ry2 = jnp.broadcast_to(load_row(sy2_ref, p), (128, 128))
            rarea = jnp.broadcast_to(load_row(area_ref, p), (128, 128))
            kp = jnp.broadcast_to(load_row(keep_ref, p), (128, 128))
            iou = _iou_tile(cx1, cy1, cx2, cy2, carea,
                            rx1, ry1, rx2, ry2, rarea)
            return jnp.maximum(
                acc, jnp.where((iou >= _THRESH) & (kp > 0.0), 1.0, 0.0))

        sup0 = lax.fori_loop(0, b, prior,
                             jnp.zeros((128, 128), jnp.float32))
        supped = jnp.max(sup0, axis=1, keepdims=True) > 0.0
        valid = jnp.where((scol > -jnp.inf) & (~supped), 1.0, 0.0)

        rx1 = jnp.broadcast_to(bx1, (128, 128))
        ry1 = jnp.broadcast_to(by1, (128, 128))
        rx2 = jnp.broadcast_to(bx2, (128, 128))
        ry2 = jnp.broadcast_to(by2, (128, 128))
        rarea = jnp.broadcast_to(barea, (128, 128))
        iou_bb = _iou_tile(cx1, cy1, cx2, cy2, carea,
                           rx1, ry1, rx2, ry2, rarea)
        lane_u = lax.broadcasted_iota(jnp.int32, (128, 128), 1)
        lane_e = lax.broadcasted_iota(jnp.int32, (128, 128), 0)
        s_mat = jnp.where((iou_bb >= _THRESH) & (lane_u < lane_e), 1.0, 0.0)

        def fix_cond(c):
            _, changed, it = c
            return changed & (it < 130)

        def fix_body(c):
            k, _, it = c
            kb = jnp.broadcast_to(jnp.transpose(k), (128, 128))
            sup = jnp.max(s_mat * kb, axis=1, keepdims=True)
            k_new = jnp.where(sup > 0.0, 0.0, valid)
            changed = jnp.sum(jnp.abs(k_new - k)) > 0.0
            return k_new, changed, it + 1

        k_fin, _, _ = lax.while_loop(
            fix_cond, fix_body,
            (valid, jnp.bool_(True), jnp.int32(0)))

        keep_ref[pl.ds(b, 1), :] = jnp.transpose(k_fin)
        return b + 1, cnt + jnp.sum(k_fin)

    def block_cond(carry):
        b, cnt = carry
        return (b < _CROWS) & (cnt < float(_POST))

    b_fin, total = lax.while_loop(block_cond, block_body,
                                  (jnp.int32(0), jnp.float32(0.0)))

    keep = keep_ref[...]
    lane_pre = keep
    for sh in (1, 2, 4, 8, 16, 32, 64):
        rolled = jnp.roll(lane_pre, sh, axis=1)
        lane_pre = lane_pre + jnp.where(lane_i >= sh, rolled, 0.0)
    rowsum = jnp.broadcast_to(jnp.sum(keep, axis=1, keepdims=True), big)
    rowcum = rowsum
    for sh in (1, 2, 4, 8, 16, 32, 64):
        rolled = jnp.roll(rowcum, sh, axis=0)
        rowcum = rowcum + jnp.where(row_i >= sh, rolled, 0.0)
    pos = (rowcum - rowsum) + lane_pre - keep
    area_ref[...] = jnp.where(keep > 0.0, pos, -1.0)

    pad0 = [jnp.broadcast_to(r[0:1, 0:1], (128, 1))
            for r in (sx1_ref, sy1_ref, sx2_ref, sy2_ref)]
    out_refs = [o0_ref, o1_ref, o2_ref, o3_ref, o4_ref]
    pay_refs = [sx1_ref, sy1_ref, sx2_ref, sy2_ref, sc_ref]
    sub_i = lax.broadcasted_iota(jnp.int32, (128, 128), 0)
    totb = jnp.full((128, 1), 0.0) + total
    for oc in range(_ORECS):
        r_col = (oc * 128 + sub_i).astype(jnp.float32)

        def omatch(j, accs):
            pe = jnp.broadcast_to(area_ref[pl.ds(j, 1), :], (128, 128))
            m = pe == r_col
            new = []
            for a, pref in zip(accs, pay_refs):
                pb = jnp.broadcast_to(pref[pl.ds(j, 1), :], (128, 128))
                new.append(a + jnp.where(m, pb, 0.0))
            return tuple(new)

        accs = tuple(jnp.zeros((128, 128), jnp.float32) for _ in range(5))
        accs = lax.fori_loop(0, b_fin, omatch, accs)
        r0 = (oc * 128 + lax.broadcasted_iota(
            jnp.int32, (128, 1), 0)).astype(jnp.float32)
        have = r0 < totb
        for c in range(5):
            col = jnp.sum(accs[c], axis=1, keepdims=True)
            pad = pad0[c] if c < 4 else jnp.full((128, 1), -jnp.inf)
            val = jnp.where(have, col, pad)
            out_refs[c][pl.ds(oc, 1), :] = jnp.transpose(val)


def _stage_c(sx1, sy1, sx2, sy2, ss):
    blk = pl.BlockSpec((_CROWS, 128), lambda i: (0, 0))
    oblk = pl.BlockSpec((_ORECS, 128), lambda i: (0, 0))
    return pl.pallas_call(
        _stage_c_body,
        grid=(1,),
        in_specs=[blk] * 5,
        out_specs=[oblk] * 5,
        out_shape=[jax.ShapeDtypeStruct((_ORECS, 128), jnp.float32)
                   for _ in range(5)],
        scratch_shapes=[pltpu.VMEM((_CROWS, 128), jnp.float32)
                        for _ in range(3)],
    )(sx1, sy1, sx2, sy2, ss)


def kernel(anchors, deltas, scores):
    pad = _NP - _N
    a_p = jnp.pad(anchors, ((0, pad), (0, 0)))
    d_p = jnp.pad(deltas, ((0, pad), (0, 0)))
    s_p = jnp.pad(scores, (0, pad))
    a_t = jnp.transpose(a_p).reshape(4, _ROWS, 128)
    d_t = jnp.transpose(d_p).reshape(4, _ROWS, 128)
    s_2d = s_p.reshape(_ROWS, 128)

    dest, x1, y1, x2, y2, ms = _stage_a(a_t, d_t, s_2d)

    c0, c1, c2, c3, c4 = _make_sc_scatter(_ROWS)(dest, x1, y1, x2, y2, ms)

    rank2 = _stage_a2(c4.reshape(_ROWS, 128))

    s0, s1, s2, s3, s4 = _make_sc_scatter(_CROWS)(
        rank2, c0.reshape(_ROWS, 128), c1.reshape(_ROWS, 128),
        c2.reshape(_ROWS, 128), c3.reshape(_ROWS, 128),
        c4.reshape(_ROWS, 128))

    outs = _stage_c(s0.reshape(_CROWS, 128), s1.reshape(_CROWS, 128),
                    s2.reshape(_CROWS, 128), s3.reshape(_CROWS, 128),
                    s4.reshape(_CROWS, 128))
    cols = [o.reshape(-1)[:_POST] for o in outs]
    return jnp.stack(cols, axis=1)


# speedup vs baseline: 100.8981x; 1.0466x over previous
import functools

import jax
import jax.numpy as jnp
from jax import lax
from jax.experimental import pallas as pl
from jax.experimental.pallas import tpu as pltpu
from jax.experimental.pallas import tpu_sc as plsc
import numpy as np

_N = 20000
_NP = 20480
_ROWS = 160
_CROWS = 96
_PRE = 12000
_POST = 2000
_ORECS = 16
_THRESH = 0.7
_IMG_W = 1024.0
_IMG_H = 1024.0
_SCALE_CLAMP = float(np.log(1000.0 / 16.0))


def _prefix_excl(x, rows):
    lane_i = lax.broadcasted_iota(jnp.int32, (rows, 128), 1)
    row_i = lax.broadcasted_iota(jnp.int32, (rows, 128), 0)
    lpre = x
    for sh in (1, 2, 4, 8, 16, 32, 64):
        lpre = lpre + jnp.where(lane_i >= sh, jnp.roll(lpre, sh, axis=1), 0.0)
    rowsum = jnp.broadcast_to(jnp.sum(x, axis=1, keepdims=True), (rows, 128))
    rcum = rowsum
    sh = 1
    while sh < rows:
        rcum = rcum + jnp.where(row_i >= sh, jnp.roll(rcum, sh, axis=0), 0.0)
        sh *= 2
    return (rcum - rowsum) + (lpre - x)


def _stage_a_body(a_ref, d_ref, s_ref, dest_ref, x1_ref, y1_ref, x2_ref,
                  y2_ref, ms_ref):
    ax1, ay1, ax2, ay2 = a_ref[0], a_ref[1], a_ref[2], a_ref[3]
    widths = ax2 - ax1
    heights = ay2 - ay1
    ctr_x = ax1 + 0.5 * widths
    ctr_y = ay1 + 0.5 * heights
    dx, dy = d_ref[0], d_ref[1]
    dw = jnp.minimum(d_ref[2], _SCALE_CLAMP)
    dh = jnp.minimum(d_ref[3], _SCALE_CLAMP)
    pred_ctr_x = dx * widths + ctr_x
    pred_ctr_y = dy * heights + ctr_y
    pred_w = jnp.exp(dw) * widths
    pred_h = jnp.exp(dh) * heights
    x1 = jnp.clip(pred_ctr_x - 0.5 * pred_w, 0.0, _IMG_W)
    y1 = jnp.clip(pred_ctr_y - 0.5 * pred_h, 0.0, _IMG_H)
    x2 = jnp.clip(pred_ctr_x + 0.5 * pred_w, 0.0, _IMG_W)
    y2 = jnp.clip(pred_ctr_y + 0.5 * pred_h, 0.0, _IMG_H)
    valid = ((x2 - x1) > 0.0) & ((y2 - y1) > 0.0)
    ms = jnp.where(valid, s_ref[...], -jnp.inf)
    ms = jnp.where(ms == 0.0, 0.0, ms)
    x1_ref[...] = x1
    y1_ref[...] = y1
    x2_ref[...] = x2
    y2_ref[...] = y2
    ms_ref[...] = ms

    bits = lax.bitcast_convert_type(ms, jnp.uint32)
    neg = (bits >> jnp.uint32(31)) == jnp.uint32(1)
    uk = jnp.where(neg, ~bits, bits | jnp.uint32(0x80000000))

    def bis(i, kacc):
        kc = kacc | lax.shift_left(jnp.uint32(1),
                                   (31 - i).astype(jnp.uint32))
        cnt = jnp.sum(jnp.where(uk >= kc, 1.0, 0.0))
        return jnp.where(cnt >= float(_PRE), kc, kacc)

    kstar = lax.fori_loop(0, 32, bis, jnp.uint32(0))

    gt = uk > kstar
    eq = uk == kstar
    cnt_gt = jnp.sum(jnp.where(gt, 1.0, 0.0))
    need = float(_PRE) - cnt_gt
    eqpre = _prefix_excl(jnp.where(eq, 1.0, 0.0), _ROWS)
    cand = gt | (eq & (eqpre < need))
    candf = jnp.where(cand, 1.0, 0.0)
    pc = _prefix_excl(candf, _ROWS)
    row_i = lax.broadcasted_iota(jnp.int32, (_ROWS, 128), 0)
    lane_i = lax.broadcasted_iota(jnp.int32, (_ROWS, 128), 1)
    flat = (row_i * 128 + lane_i).astype(jnp.float32)
    dest = jnp.where(cand, pc, float(_PRE) + flat - pc)
    dest_ref[...] = dest.astype(jnp.int32)


def _stage_a(anchors_t, deltas_t, scores_p):
    return pl.pallas_call(
        _stage_a_body,
        out_shape=(
            jax.ShapeDtypeStruct((_ROWS, 128), jnp.int32),
            jax.ShapeDtypeStruct((_ROWS, 128), jnp.float32),
            jax.ShapeDtypeStruct((_ROWS, 128), jnp.float32),
            jax.ShapeDtypeStruct((_ROWS, 128), jnp.float32),
            jax.ShapeDtypeStruct((_ROWS, 128), jnp.float32),
            jax.ShapeDtypeStruct((_ROWS, 128), jnp.float32),
        ),
    )(anchors_t, deltas_t, scores_p)


def _stage_a2_body(ms_ref, rank_ref):
    def outer(r, carry):
        trow = ms_ref[pl.ds(r, 1), :]
        tb = jnp.broadcast_to(jnp.transpose(trow), (128, 128))

        def body_ge2(i, acc):
            cb = jnp.broadcast_to(ms_ref[pl.ds(2 * i, 1), :], (128, 128))
            acc = acc + (cb >= tb).astype(jnp.float32)
            cb = jnp.broadcast_to(ms_ref[pl.ds(2 * i + 1, 1), :], (128, 128))
            return acc + (cb >= tb).astype(jnp.float32)

        def body_gt2(i, acc):
            base = r + 1 + 2 * i
            cb = jnp.broadcast_to(ms_ref[pl.ds(base, 1), :], (128, 128))
            acc = acc + (cb > tb).astype(jnp.float32)
            cb = jnp.broadcast_to(ms_ref[pl.ds(base + 1, 1), :], (128, 128))
            return acc + (cb > tb).astype(jnp.float32)

        acc = jnp.zeros((128, 128), jnp.float32)
        acc = lax.fori_loop(0, r // 2, body_ge2, acc)
        tail = jnp.broadcast_to(
            ms_ref[pl.ds(jnp.maximum(r - 1, 0), 1), :], (128, 128))
        acc = acc + jnp.where((r % 2) == 1,
                              (tail >= tb).astype(jnp.float32), 0.0)
        ng = (_CROWS - 1) - r
        acc = lax.fori_loop(0, ng // 2, body_gt2, acc)
        tail2 = jnp.broadcast_to(ms_ref[pl.ds(_CROWS - 1, 1), :], (128, 128))
        acc = acc + jnp.where((ng % 2) == 1,
                              (tail2 > tb).astype(jnp.float32), 0.0)
        cb = jnp.broadcast_to(trow, (128, 128))
        lane_j = lax.broadcasted_iota(jnp.int32, (128, 128), 1)
        lane_t = lax.broadcasted_iota(jnp.int32, (128, 128), 0)
        acc = acc + (cb > tb).astype(jnp.float32)
        acc = acc + ((cb == tb) & (lane_j < lane_t)).astype(jnp.float32)
        cnt = jnp.sum(acc, axis=1, keepdims=True)
        rank_ref[pl.ds(r, 1), :] = jnp.transpose(cnt).astype(jnp.int32)
        return carry

    lax.fori_loop(0, _CROWS, outer, 0)


def _stage_a2(cms):
    blk = pl.BlockSpec((_CROWS, 128), lambda i: (0, 0))
    oblk = pl.BlockSpec((_CROWS, 128), lambda i: (0, 0))
    return pl.pallas_call(
        _stage_a2_body,
        grid=(1,),
        in_specs=[blk],
        out_specs=oblk,
        out_shape=jax.ShapeDtypeStruct((_CROWS, 128), jnp.int32),
    )(cms)


def _make_sc_scatter(n_rows):
    mesh = plsc.VectorSubcoreMesh(core_axis_name="c", subcore_axis_name="s")
    chunk_rows = 8
    n_chunks = n_rows // chunk_rows
    n_out = n_rows * 128
    out_elems = n_out // 16

    @functools.partial(
        pl.kernel,
        mesh=mesh,
        out_type=[jax.ShapeDtypeStruct((n_out,), jnp.float32)
                  for _ in range(5)],
        scratch_types=(
            [pltpu.VMEM((chunk_rows, 128), jnp.int32)]
            + [pltpu.VMEM((chunk_rows, 128), jnp.float32)
               for _ in range(5)]
            + [pltpu.VMEM_SHARED((n_out,), jnp.float32) for _ in range(5)]
            + [pltpu.SemaphoreType.DMA]
        ),
    )
    def sc_scatter(rank_hbm, x1h, y1h, x2h, y2h, msh,
                   o0, o1, o2, o3, o4,
                   idx_v, v0, v1, v2, v3, v4,
                   sh0, sh1, sh2, sh3, sh4, sem):
        core = lax.axis_index("c")
        tec = lax.axis_index("s")
        ins = [x1h, y1h, x2h, y2h, msh]
        stages = [v0, v1, v2, v3, v4]
        shared = [sh0, sh1, sh2, sh3, sh4]
        outs = [o0, o1, o2, o3, o4]

        @pl.when(core == 0)
        def _():
            def do_chunk(k):
                base = k * chunk_rows
                pltpu.sync_copy(rank_hbm.at[pl.ds(base, chunk_rows), :],
                                idx_v)
                for c in range(5):
                    pltpu.sync_copy(ins[c].at[pl.ds(base, chunk_rows), :],
                                    stages[c])
                descs = []
                for c in range(5):
                    for j in range(chunk_rows):
                        descs.append(
                            pltpu.async_copy(stages[c].at[j],
                                             shared[c].at[idx_v.at[j]],
                                             sem))
                for d in descs:
                    d.wait()

            @pl.when(tec < n_chunks)
            def _():
                do_chunk(tec)

            @pl.when(tec < n_chunks - 16)
            def _():
                do_chunk(tec + 16)

        plsc.subcore_barrier()

        @pl.when(core == 0)
        def _():
            base = tec * out_elems
            for c in range(5):
                pltpu.sync_copy(shared[c].at[pl.ds(base, out_elems)],
                                outs[c].at[pl.ds(base, out_elems)])

    return sc_scatter


def _iou_tile(cx1, cy1, cx2, cy2, carea, rx1, ry1, rx2, ry2, rarea):
    ix1 = jnp.maximum(rx1, cx1)
    iy1 = jnp.maximum(ry1, cy1)
    ix2 = jnp.minimum(rx2, cx2)
    iy2 = jnp.minimum(ry2, cy2)
    iw = jnp.maximum(ix2 - ix1, 0.0)
    ih = jnp.maximum(iy2 - iy1, 0.0)
    inter = iw * ih
    return inter / (rarea + carea - inter + 1e-9)


def _stage_c_body(sx1_ref, sy1_ref, sx2_ref, sy2_ref, ss_ref,
                  o0_ref, o1_ref, o2_ref, o3_ref, o4_ref,
                  keep_ref, sc_ref, area_ref):
    big = (_CROWS, 128)
    row_i = lax.broadcasted_iota(jnp.int32, big, 0)
    lane_i = lax.broadcasted_iota(jnp.int32, big, 1)
    flat = row_i * 128 + lane_i
    sc_ref[...] = jnp.where(flat < _PRE, ss_ref[...], -jnp.inf)
    x1a, y1a = sx1_ref[...], sy1_ref[...]
    x2a, y2a = sx2_ref[...], sy2_ref[...]
    area_ref[...] = (jnp.maximum(x2a - x1a, 0.0)
                     * jnp.maximum(y2a - y1a, 0.0))
    keep_ref[...] = jnp.zeros(big, jnp.float32)

    def load_row(ref, i):
        return ref[pl.ds(i, 1), :]

    def block_body(carry):
        b, cnt = carry
        bx1 = load_row(sx1_ref, b)
        by1 = load_row(sy1_ref, b)
        bx2 = load_row(sx2_ref, b)
        by2 = load_row(sy2_ref, b)
        barea = load_row(area_ref, b)
        bs = load_row(sc_ref, b)
        cx1 = jnp.broadcast_to(jnp.transpose(bx1), (128, 128))
        cy1 = jnp.broadcast_to(jnp.transpose(by1), (128, 128))
        cx2 = jnp.broadcast_to(jnp.transpose(bx2), (128, 128))
        cy2 = jnp.broadcast_to(jnp.transpose(by2), (128, 128))
        carea = jnp.broadcast_to(jnp.transpose(barea), (128, 128))
        scol = jnp.transpose(bs)

        def prior(p, acc):
            rx1 = jnp.broadcast_to(load_row(sx1_ref, p), (128, 128))
            ry1 = jnp.broadcast_to(load_row(sy1_ref, p), (128, 128))
            rx2 = jnp.broadcast_to(load_row(sx2_ref, p), (128, 128))
            ry2 = jnp.broadcast_to(load_row(sy2_ref, p), (128, 128))
            rarea = jnp.broadcast_to(load_row(area_ref, p), (128, 128))
            kp = jnp.broadcast_to(load_row(keep_ref, p), (128, 128))
            iou = _iou_tile(cx1, cy1, cx2, cy2, carea,
                            rx1, ry1, rx2, ry2, rarea)
            return jnp.maximum(
                acc, jnp.where((iou >= _THRESH) & (kp > 0.0), 1.0, 0.0))

        sup0 = lax.fori_loop(0, b, prior,
                             jnp.zeros((128, 128), jnp.float32))
        supped = jnp.max(sup0, axis=1, keepdims=True) > 0.0
        valid = jnp.where((scol > -jnp.inf) & (~supped), 1.0, 0.0)

        rx1 = jnp.broadcast_to(bx1, (128, 128))
        ry1 = jnp.broadcast_to(by1, (128, 128))
        rx2 = jnp.broadcast_to(bx2, (128, 128))
        ry2 = jnp.broadcast_to(by2, (128, 128))
        rarea = jnp.broadcast_to(barea, (128, 128))
        iou_bb = _iou_tile(cx1, cy1, cx2, cy2, carea,
                           rx1, ry1, rx2, ry2, rarea)
        lane_u = lax.broadcasted_iota(jnp.int32, (128, 128), 1)
        lane_e = lax.broadcasted_iota(jnp.int32, (128, 128), 0)
        s_mat = jnp.where((iou_bb >= _THRESH) & (lane_u < lane_e), 1.0, 0.0)

        def fix_cond(c):
            _, changed, it = c
            return changed & (it < 130)

        def fix_body(c):
            k, _, it = c
            kb = jnp.broadcast_to(jnp.transpose(k), (128, 128))
            sup = jnp.max(s_mat * kb, axis=1, keepdims=True)
            k_new = jnp.where(sup > 0.0, 0.0, valid)
            changed = jnp.sum(jnp.abs(k_new - k)) > 0.0
            return k_new, changed, it + 1

        k_fin, _, _ = lax.while_loop(
            fix_cond, fix_body,
            (valid, jnp.bool_(True), jnp.int32(0)))

        keep_ref[pl.ds(b, 1), :] = jnp.transpose(k_fin)
        return b + 1, cnt + jnp.sum(k_fin)

    def block_cond(carry):
        b, cnt = carry
        return (b < _CROWS) & (cnt < float(_POST))

    b_fin, total = lax.while_loop(block_cond, block_body,
                                  (jnp.int32(0), jnp.float32(0.0)))

    keep = keep_ref[...]
    lane_pre = keep
    for sh in (1, 2, 4, 8, 16, 32, 64):
        rolled = jnp.roll(lane_pre, sh, axis=1)
        lane_pre = lane_pre + jnp.where(lane_i >= sh, rolled, 0.0)
    rowsum = jnp.broadcast_to(jnp.sum(keep, axis=1, keepdims=True), big)
    rowcum = rowsum
    for sh in (1, 2, 4, 8, 16, 32, 64):
        rolled = jnp.roll(rowcum, sh, axis=0)
        rowcum = rowcum + jnp.where(row_i >= sh, rolled, 0.0)
    pos = (rowcum - rowsum) + lane_pre - keep
    area_ref[...] = jnp.where(keep > 0.0, pos, -1.0)

    pad0 = [jnp.broadcast_to(r[0:1, 0:1], (128, 1))
            for r in (sx1_ref, sy1_ref, sx2_ref, sy2_ref)]
    out_refs = [o0_ref, o1_ref, o2_ref, o3_ref, o4_ref]
    pay_refs = [sx1_ref, sy1_ref, sx2_ref, sy2_ref, sc_ref]
    sub_i = lax.broadcasted_iota(jnp.int32, (128, 128), 0)
    totb = jnp.full((128, 1), 0.0) + total
    for oc in range(_ORECS):
        r_col = (oc * 128 + sub_i).astype(jnp.float32)

        def omatch(j, accs):
            pe = jnp.broadcast_to(area_ref[pl.ds(j, 1), :], (128, 128))
            m = pe == r_col
            new = []
            for a, pref in zip(accs, pay_refs):
                pb = jnp.broadcast_to(pref[pl.ds(j, 1), :], (128, 128))
                new.append(a + jnp.where(m, pb, 0.0))
            return tuple(new)

        accs = tuple(jnp.zeros((128, 128), jnp.float32) for _ in range(5))
        accs = lax.fori_loop(0, b_fin, omatch, accs)
        r0 = (oc * 128 + lax.broadcasted_iota(
            jnp.int32, (128, 1), 0)).astype(jnp.float32)
        have = r0 < totb
        for c in range(5):
            col = jnp.sum(accs[c], axis=1, keepdims=True)
            pad = pad0[c] if c < 4 else jnp.full((128, 1), -jnp.inf)
            val = jnp.where(have, col, pad)
            out_refs[c][pl.ds(oc, 1), :] = jnp.transpose(val)


def _stage_c(sx1, sy1, sx2, sy2, ss):
    blk = pl.BlockSpec((_CROWS, 128), lambda i: (0, 0))
    oblk = pl.BlockSpec((_ORECS, 128), lambda i: (0, 0))
    return pl.pallas_call(
        _stage_c_body,
        grid=(1,),
        in_specs=[blk] * 5,
        out_specs=[oblk] * 5,
        out_shape=[jax.ShapeDtypeStruct((_ORECS, 128), jnp.float32)
                   for _ in range(5)],
        scratch_shapes=[pltpu.VMEM((_CROWS, 128), jnp.float32)
                        for _ in range(3)],
    )(sx1, sy1, sx2, sy2, ss)


def kernel(anchors, deltas, scores):
    pad = _NP - _N
    a_p = jnp.pad(anchors, ((0, pad), (0, 0)))
    d_p = jnp.pad(deltas, ((0, pad), (0, 0)))
    s_p = jnp.pad(scores, (0, pad))
    a_t = jnp.transpose(a_p).reshape(4, _ROWS, 128)
    d_t = jnp.transpose(d_p).reshape(4, _ROWS, 128)
    s_2d = s_p.reshape(_ROWS, 128)

    dest, x1, y1, x2, y2, ms = _stage_a(a_t, d_t, s_2d)

    c0, c1, c2, c3, c4 = _make_sc_scatter(_ROWS)(dest, x1, y1, x2, y2, ms)

    rank2 = _stage_a2(c4.reshape(_ROWS, 128))

    s0, s1, s2, s3, s4 = _make_sc_scatter(_CROWS)(
        rank2, c0.reshape(_ROWS, 128), c1.reshape(_ROWS, 128),
        c2.reshape(_ROWS, 128), c3.reshape(_ROWS, 128),
        c4.reshape(_ROWS, 128))

    outs = _stage_c(s0.reshape(_CROWS, 128), s1.reshape(_CROWS, 128),
                    s2.reshape(_CROWS, 128), s3.reshape(_CROWS, 128),
                    s4.reshape(_CROWS, 128))
    cols = [o.reshape(-1)[:_POST] for o in outs]
    return jnp.stack(cols, axis=1)


# speedup vs baseline: 106.3247x; 1.0538x over previous
import functools

import jax
import jax.numpy as jnp
from jax import lax
from jax.experimental import pallas as pl
from jax.experimental.pallas import tpu as pltpu
from jax.experimental.pallas import tpu_sc as plsc
import numpy as np

_N = 20000
_NP = 20480
_ROWS = 160
_CROWS = 96
_PRE = 12000
_POST = 2000
_ORECS = 16
_THRESH = 0.7
_IMG_W = 1024.0
_IMG_H = 1024.0
_SCALE_CLAMP = float(np.log(1000.0 / 16.0))


def _prefix_excl(x, rows):
    lane_i = lax.broadcasted_iota(jnp.int32, (rows, 128), 1)
    row_i = lax.broadcasted_iota(jnp.int32, (rows, 128), 0)
    lpre = x
    for sh in (1, 2, 4, 8, 16, 32, 64):
        lpre = lpre + jnp.where(lane_i >= sh, jnp.roll(lpre, sh, axis=1), 0.0)
    rowsum = jnp.broadcast_to(jnp.sum(x, axis=1, keepdims=True), (rows, 128))
    rcum = rowsum
    sh = 1
    while sh < rows:
        rcum = rcum + jnp.where(row_i >= sh, jnp.roll(rcum, sh, axis=0), 0.0)
        sh *= 2
    return (rcum - rowsum) + (lpre - x)


def _stage_a_body(a_ref, d_ref, s_ref, dest_ref, x1_ref, y1_ref, x2_ref,
                  y2_ref, ms_ref):
    ax1, ay1, ax2, ay2 = a_ref[0], a_ref[1], a_ref[2], a_ref[3]
    widths = ax2 - ax1
    heights = ay2 - ay1
    ctr_x = ax1 + 0.5 * widths
    ctr_y = ay1 + 0.5 * heights
    dx, dy = d_ref[0], d_ref[1]
    dw = jnp.minimum(d_ref[2], _SCALE_CLAMP)
    dh = jnp.minimum(d_ref[3], _SCALE_CLAMP)
    pred_ctr_x = dx * widths + ctr_x
    pred_ctr_y = dy * heights + ctr_y
    pred_w = jnp.exp(dw) * widths
    pred_h = jnp.exp(dh) * heights
    x1 = jnp.clip(pred_ctr_x - 0.5 * pred_w, 0.0, _IMG_W)
    y1 = jnp.clip(pred_ctr_y - 0.5 * pred_h, 0.0, _IMG_H)
    x2 = jnp.clip(pred_ctr_x + 0.5 * pred_w, 0.0, _IMG_W)
    y2 = jnp.clip(pred_ctr_y + 0.5 * pred_h, 0.0, _IMG_H)
    valid = ((x2 - x1) > 0.0) & ((y2 - y1) > 0.0)
    ms = jnp.where(valid, s_ref[...], -jnp.inf)
    ms = jnp.where(ms == 0.0, 0.0, ms)
    x1_ref[...] = x1
    y1_ref[...] = y1
    x2_ref[...] = x2
    y2_ref[...] = y2
    ms_ref[...] = ms

    bits = lax.bitcast_convert_type(ms, jnp.uint32)
    neg = (bits >> jnp.uint32(31)) == jnp.uint32(1)
    uk = jnp.where(neg, ~bits, bits | jnp.uint32(0x80000000))

    def bis(i, kacc):
        kc = kacc | lax.shift_left(jnp.uint32(1),
                                   (31 - i).astype(jnp.uint32))
        cnt = jnp.sum(jnp.where(uk >= kc, 1.0, 0.0))
        return jnp.where(cnt >= float(_PRE), kc, kacc)

    kstar = lax.fori_loop(0, 32, bis, jnp.uint32(0))

    gt = uk > kstar
    eq = uk == kstar
    cnt_gt = jnp.sum(jnp.where(gt, 1.0, 0.0))
    need = float(_PRE) - cnt_gt
    eqpre = _prefix_excl(jnp.where(eq, 1.0, 0.0), _ROWS)
    cand = gt | (eq & (eqpre < need))
    candf = jnp.where(cand, 1.0, 0.0)
    pc = _prefix_excl(candf, _ROWS)
    row_i = lax.broadcasted_iota(jnp.int32, (_ROWS, 128), 0)
    lane_i = lax.broadcasted_iota(jnp.int32, (_ROWS, 128), 1)
    flat = (row_i * 128 + lane_i).astype(jnp.float32)
    dest = jnp.where(cand, pc, float(_PRE) + flat - pc)
    dest_ref[...] = dest.astype(jnp.int32)


def _stage_a(anchors_t, deltas_t, scores_p):
    return pl.pallas_call(
        _stage_a_body,
        out_shape=(
            jax.ShapeDtypeStruct((_ROWS, 128), jnp.int32),
            jax.ShapeDtypeStruct((_ROWS, 128), jnp.float32),
            jax.ShapeDtypeStruct((_ROWS, 128), jnp.float32),
            jax.ShapeDtypeStruct((_ROWS, 128), jnp.float32),
            jax.ShapeDtypeStruct((_ROWS, 128), jnp.float32),
            jax.ShapeDtypeStruct((_ROWS, 128), jnp.float32),
        ),
    )(anchors_t, deltas_t, scores_p)


def _stage_a2_body(ms_ref, rank_ref):
    def outer(r, carry):
        trow = ms_ref[pl.ds(r, 1), :]
        tb = jnp.broadcast_to(jnp.transpose(trow), (128, 128))

        def body_ge2(i, acc):
            cb = jnp.broadcast_to(ms_ref[pl.ds(2 * i, 1), :], (128, 128))
            acc = acc + (cb >= tb).astype(jnp.float32)
            cb = jnp.broadcast_to(ms_ref[pl.ds(2 * i + 1, 1), :], (128, 128))
            return acc + (cb >= tb).astype(jnp.float32)

        def body_gt2(i, acc):
            base = r + 1 + 2 * i
            cb = jnp.broadcast_to(ms_ref[pl.ds(base, 1), :], (128, 128))
            acc = acc + (cb > tb).astype(jnp.float32)
            cb = jnp.broadcast_to(ms_ref[pl.ds(base + 1, 1), :], (128, 128))
            return acc + (cb > tb).astype(jnp.float32)

        acc = jnp.zeros((128, 128), jnp.float32)
        acc = lax.fori_loop(0, r // 2, body_ge2, acc)
        tail = jnp.broadcast_to(
            ms_ref[pl.ds(jnp.maximum(r - 1, 0), 1), :], (128, 128))
        acc = acc + jnp.where((r % 2) == 1,
                              (tail >= tb).astype(jnp.float32), 0.0)
        ng = (_CROWS - 1) - r
        acc = lax.fori_loop(0, ng // 2, body_gt2, acc)
        tail2 = jnp.broadcast_to(ms_ref[pl.ds(_CROWS - 1, 1), :], (128, 128))
        acc = acc + jnp.where((ng % 2) == 1,
                              (tail2 > tb).astype(jnp.float32), 0.0)
        cb = jnp.broadcast_to(trow, (128, 128))
        lane_j = lax.broadcasted_iota(jnp.int32, (128, 128), 1)
        lane_t = lax.broadcasted_iota(jnp.int32, (128, 128), 0)
        acc = acc + (cb > tb).astype(jnp.float32)
        acc = acc + ((cb == tb) & (lane_j < lane_t)).astype(jnp.float32)
        cnt = jnp.sum(acc, axis=1, keepdims=True)
        rank_ref[pl.ds(r, 1), :] = jnp.transpose(cnt).astype(jnp.int32)
        return carry

    lax.fori_loop(0, _CROWS, outer, 0)


def _stage_a2(cms):
    blk = pl.BlockSpec((_CROWS, 128), lambda i: (0, 0))
    oblk = pl.BlockSpec((_CROWS, 128), lambda i: (0, 0))
    return pl.pallas_call(
        _stage_a2_body,
        grid=(1,),
        in_specs=[blk],
        out_specs=oblk,
        out_shape=jax.ShapeDtypeStruct((_CROWS, 128), jnp.int32),
    )(cms)


def _make_sc_scatter(n_rows, n_arr=5, on_core=0):
    mesh = plsc.VectorSubcoreMesh(core_axis_name="c", subcore_axis_name="s")
    chunk_rows = 8
    n_chunks = n_rows // chunk_rows
    n_out = n_rows * 128
    out_elems = n_out // 16

    @functools.partial(
        pl.kernel,
        mesh=mesh,
        out_type=[jax.ShapeDtypeStruct((n_out,), jnp.float32)
                  for _ in range(n_arr)],
        scratch_types=(
            [pltpu.VMEM((chunk_rows, 128), jnp.int32)]
            + [pltpu.VMEM((chunk_rows, 128), jnp.float32)
               for _ in range(n_arr)]
            + [pltpu.VMEM_SHARED((n_out,), jnp.float32)
               for _ in range(n_arr)]
            + [pltpu.SemaphoreType.DMA]
        ),
    )
    def sc_scatter(*refs):
        rank_hbm = refs[0]
        ins = refs[1:1 + n_arr]
        outs = refs[1 + n_arr:1 + 2 * n_arr]
        idx_v = refs[1 + 2 * n_arr]
        stages = refs[2 + 2 * n_arr:2 + 3 * n_arr]
        shared = refs[2 + 3 * n_arr:2 + 4 * n_arr]
        sem = refs[-1]
        core = lax.axis_index("c")
        tec = lax.axis_index("s")

        @pl.when(core == on_core)
        def _():
            def do_chunk(k):
                base = k * chunk_rows
                pltpu.sync_copy(rank_hbm.at[pl.ds(base, chunk_rows), :],
                                idx_v)
                for c in range(n_arr):
                    pltpu.sync_copy(ins[c].at[pl.ds(base, chunk_rows), :],
                                    stages[c])
                descs = []
                for c in range(n_arr):
                    for j in range(chunk_rows):
                        descs.append(
                            pltpu.async_copy(stages[c].at[j],
                                             shared[c].at[idx_v.at[j]],
                                             sem))
                for d in descs:
                    d.wait()

            @pl.when(tec < n_chunks)
            def _():
                do_chunk(tec)

            @pl.when(tec < n_chunks - 16)
            def _():
                do_chunk(tec + 16)

        plsc.subcore_barrier()

        @pl.when(core == on_core)
        def _():
            base = tec * out_elems
            for c in range(n_arr):
                pltpu.sync_copy(shared[c].at[pl.ds(base, out_elems)],
                                outs[c].at[pl.ds(base, out_elems)])

    return sc_scatter


def _iou_tile(cx1, cy1, cx2, cy2, carea, rx1, ry1, rx2, ry2, rarea):
    ix1 = jnp.maximum(rx1, cx1)
    iy1 = jnp.maximum(ry1, cy1)
    ix2 = jnp.minimum(rx2, cx2)
    iy2 = jnp.minimum(ry2, cy2)
    iw = jnp.maximum(ix2 - ix1, 0.0)
    ih = jnp.maximum(iy2 - iy1, 0.0)
    inter = iw * ih
    return inter / (rarea + carea - inter + 1e-9)


def _stage_c_body(sx1_ref, sy1_ref, sx2_ref, sy2_ref, ss_ref,
                  o0_ref, o1_ref, o2_ref, o3_ref, o4_ref,
                  keep_ref, sc_ref, area_ref):
    big = (_CROWS, 128)
    row_i = lax.broadcasted_iota(jnp.int32, big, 0)
    lane_i = lax.broadcasted_iota(jnp.int32, big, 1)
    flat = row_i * 128 + lane_i
    sc_ref[...] = jnp.where(flat < _PRE, ss_ref[...], -jnp.inf)
    x1a, y1a = sx1_ref[...], sy1_ref[...]
    x2a, y2a = sx2_ref[...], sy2_ref[...]
    area_ref[...] = (jnp.maximum(x2a - x1a, 0.0)
                     * jnp.maximum(y2a - y1a, 0.0))
    keep_ref[...] = jnp.zeros(big, jnp.float32)

    def load_row(ref, i):
        return ref[pl.ds(i, 1), :]

    def block_body(carry):
        b, cnt = carry
        bx1 = load_row(sx1_ref, b)
        by1 = load_row(sy1_ref, b)
        bx2 = load_row(sx2_ref, b)
        by2 = load_row(sy2_ref, b)
        barea = load_row(area_ref, b)
        bs = load_row(sc_ref, b)
        cx1 = jnp.broadcast_to(jnp.transpose(bx1), (128, 128))
        cy1 = jnp.broadcast_to(jnp.transpose(by1), (128, 128))
        cx2 = jnp.broadcast_to(jnp.transpose(bx2), (128, 128))
        cy2 = jnp.broadcast_to(jnp.transpose(by2), (128, 128))
        carea = jnp.broadcast_to(jnp.transpose(barea), (128, 128))
        scol = jnp.transpose(bs)

        def prior(p, acc):
            rx1 = jnp.broadcast_to(load_row(sx1_ref, p), (128, 128))
            ry1 = jnp.broadcast_to(load_row(sy1_ref, p), (128, 128))
            rx2 = jnp.broadcast_to(load_row(sx2_ref, p), (128, 128))
            ry2 = jnp.broadcast_to(load_row(sy2_ref, p), (128, 128))
            rarea = jnp.broadcast_to(load_row(area_ref, p), (128, 128))
            kp = jnp.broadcast_to(load_row(keep_ref, p), (128, 128))
            iou = _iou_tile(cx1, cy1, cx2, cy2, carea,
                            rx1, ry1, rx2, ry2, rarea)
            return jnp.maximum(
                acc, jnp.where((iou >= _THRESH) & (kp > 0.0), 1.0, 0.0))

        sup0 = lax.fori_loop(0, b, prior,
                             jnp.zeros((128, 128), jnp.float32))
        supped = jnp.max(sup0, axis=1, keepdims=True) > 0.0
        valid = jnp.where((scol > -jnp.inf) & (~supped), 1.0, 0.0)

        rx1 = jnp.broadcast_to(bx1, (128, 128))
        ry1 = jnp.broadcast_to(by1, (128, 128))
        rx2 = jnp.broadcast_to(bx2, (128, 128))
        ry2 = jnp.broadcast_to(by2, (128, 128))
        rarea = jnp.broadcast_to(barea, (128, 128))
        iou_bb = _iou_tile(cx1, cy1, cx2, cy2, carea,
                           rx1, ry1, rx2, ry2, rarea)
        lane_u = lax.broadcasted_iota(jnp.int32, (128, 128), 1)
        lane_e = lax.broadcasted_iota(jnp.int32, (128, 128), 0)
        s_mat = jnp.where((iou_bb >= _THRESH) & (lane_u < lane_e), 1.0, 0.0)

        def fix_cond(c):
            _, changed, it = c
            return changed & (it < 130)

        def fix_body(c):
            k, _, it = c
            kb = jnp.broadcast_to(jnp.transpose(k), (128, 128))
            sup = jnp.max(s_mat * kb, axis=1, keepdims=True)
            k_new = jnp.where(sup > 0.0, 0.0, valid)
            changed = jnp.sum(jnp.abs(k_new - k)) > 0.0
            return k_new, changed, it + 1

        k_fin, _, _ = lax.while_loop(
            fix_cond, fix_body,
            (valid, jnp.bool_(True), jnp.int32(0)))

        keep_ref[pl.ds(b, 1), :] = jnp.transpose(k_fin)
        return b + 1, cnt + jnp.sum(k_fin)

    def block_cond(carry):
        b, cnt = carry
        return (b < _CROWS) & (cnt < float(_POST))

    b_fin, total = lax.while_loop(block_cond, block_body,
                                  (jnp.int32(0), jnp.float32(0.0)))

    keep = keep_ref[...]
    lane_pre = keep
    for sh in (1, 2, 4, 8, 16, 32, 64):
        rolled = jnp.roll(lane_pre, sh, axis=1)
        lane_pre = lane_pre + jnp.where(lane_i >= sh, rolled, 0.0)
    rowsum = jnp.broadcast_to(jnp.sum(keep, axis=1, keepdims=True), big)
    rowcum = rowsum
    for sh in (1, 2, 4, 8, 16, 32, 64):
        rolled = jnp.roll(rowcum, sh, axis=0)
        rowcum = rowcum + jnp.where(row_i >= sh, rolled, 0.0)
    pos = (rowcum - rowsum) + lane_pre - keep
    area_ref[...] = jnp.where(keep > 0.0, pos, -1.0)

    pad0 = [jnp.broadcast_to(r[0:1, 0:1], (128, 1))
            for r in (sx1_ref, sy1_ref, sx2_ref, sy2_ref)]
    out_refs = [o0_ref, o1_ref, o2_ref, o3_ref, o4_ref]
    pay_refs = [sx1_ref, sy1_ref, sx2_ref, sy2_ref, sc_ref]
    sub_i = lax.broadcasted_iota(jnp.int32, (128, 128), 0)
    totb = jnp.full((128, 1), 0.0) + total
    for oc in range(_ORECS):
        r_col = (oc * 128 + sub_i).astype(jnp.float32)

        def omatch(j, accs):
            pe = jnp.broadcast_to(area_ref[pl.ds(j, 1), :], (128, 128))
            m = pe == r_col
            new = []
            for a, pref in zip(accs, pay_refs):
                pb = jnp.broadcast_to(pref[pl.ds(j, 1), :], (128, 128))
                new.append(a + jnp.where(m, pb, 0.0))
            return tuple(new)

        accs = tuple(jnp.zeros((128, 128), jnp.float32) for _ in range(5))
        accs = lax.fori_loop(0, b_fin, omatch, accs)
        r0 = (oc * 128 + lax.broadcasted_iota(
            jnp.int32, (128, 1), 0)).astype(jnp.float32)
        have = r0 < totb
        for c in range(5):
            col = jnp.sum(accs[c], axis=1, keepdims=True)
            pad = pad0[c] if c < 4 else jnp.full((128, 1), -jnp.inf)
            val = jnp.where(have, col, pad)
            out_refs[c][pl.ds(oc, 1), :] = jnp.transpose(val)


def _stage_c(sx1, sy1, sx2, sy2, ss):
    blk = pl.BlockSpec((_CROWS, 128), lambda i: (0, 0))
    oblk = pl.BlockSpec((_ORECS, 128), lambda i: (0, 0))
    return pl.pallas_call(
        _stage_c_body,
        grid=(1,),
        in_specs=[blk] * 5,
        out_specs=[oblk] * 5,
        out_shape=[jax.ShapeDtypeStruct((_ORECS, 128), jnp.float32)
                   for _ in range(5)],
        scratch_shapes=[pltpu.VMEM((_CROWS, 128), jnp.float32)
                        for _ in range(3)],
    )(sx1, sy1, sx2, sy2, ss)


def kernel(anchors, deltas, scores):
    pad = _NP - _N
    a_p = jnp.pad(anchors, ((0, pad), (0, 0)))
    d_p = jnp.pad(deltas, ((0, pad), (0, 0)))
    s_p = jnp.pad(scores, (0, pad))
    a_t = jnp.transpose(a_p).reshape(4, _ROWS, 128)
    d_t = jnp.transpose(d_p).reshape(4, _ROWS, 128)
    s_2d = s_p.reshape(_ROWS, 128)

    dest, x1, y1, x2, y2, ms = _stage_a(a_t, d_t, s_2d)

    (c4,) = _make_sc_scatter(_ROWS, n_arr=1, on_core=0)(dest, ms)
    c0, c1, c2, c3 = _make_sc_scatter(_ROWS, n_arr=4, on_core=1)(
        dest, x1, y1, x2, y2)

    rank2 = _stage_a2(c4.reshape(_ROWS, 128))

    s0, s1, s2, s3, s4 = _make_sc_scatter(_CROWS)(
        rank2, c0.reshape(_ROWS, 128), c1.reshape(_ROWS, 128),
        c2.reshape(_ROWS, 128), c3.reshape(_ROWS, 128),
        c4.reshape(_ROWS, 128))

    outs = _stage_c(s0.reshape(_CROWS, 128), s1.reshape(_CROWS, 128),
                    s2.reshape(_CROWS, 128), s3.reshape(_CROWS, 128),
                    s4.reshape(_CROWS, 128))
    cols = [o.reshape(-1)[:_POST] for o in outs]
    return jnp.stack(cols, axis=1)
